# baseline (speedup 1.0000x reference)
import functools

import jax
import jax.numpy as jnp
from jax import lax
from jax.experimental import pallas as pl
from jax.experimental.pallas import tpu as pltpu
from jax.experimental.pallas import tpu_sc as plsc

_TOKENS = 4096
_D = 1024
_E = 16
_CHUNK = 128
_TAU = 0.006
_NW = 32
_TPW = _TOKENS // _NW
_K = 16
_MAXU = _NW * _K
_BTA = 512
_BTC = 64
_BIG = 1 << 20


def _top2_masks(logits):
    iota = lax.broadcasted_iota(jnp.int32, logits.shape, 1)
    m1 = jnp.max(logits, axis=1, keepdims=True)
    i1 = jnp.min(jnp.where(logits == m1, iota, _E), axis=1, keepdims=True)
    sel1 = iota == i1
    masked = jnp.where(sel1, -jnp.inf, logits)
    m2 = jnp.max(masked, axis=1, keepdims=True)
    i2 = jnp.min(jnp.where(masked == m2, iota, _E), axis=1, keepdims=True)
    sel2 = iota == i2
    return sel1, sel2, m1, m2


def _softmax2_dense(sel1, sel2, m1, m2):
    q = jnp.exp(m2 - m1)
    g1 = 1.0 / (1.0 + q)
    g2 = q / (1.0 + q)
    return jnp.where(sel1, g1, 0.0) + jnp.where(sel2, g2, 0.0)


def _approx_block(t1_ref, t2_ref, w1t_ref, w2t_ref, dense_ref, idx_ref, src_ref):
    t1 = t1_ref[...]
    t2 = t2_ref[...]
    w1t = w1t_ref[...]
    w2t = w2t_ref[...]
    hi = jax.lax.Precision.HIGHEST
    dot = (jnp.dot(t1, w1t, precision=hi, preferred_element_type=jnp.float32)
           + jnp.dot(t2, w2t, precision=hi, preferred_element_type=jnp.float32))
    xsq = (jnp.sum(t1 * t1, axis=1, keepdims=True)
           + jnp.sum(t2 * t2, axis=1, keepdims=True))
    wsq = (jnp.sum(w1t * w1t, axis=0, keepdims=True)
           + jnp.sum(w2t * w2t, axis=0, keepdims=True))
    d2 = xsq + wsq - 2.0 * dot

    iota = lax.broadcasted_iota(jnp.int32, d2.shape, 1)
    inf = jnp.float32(jnp.inf)
    m1 = jnp.min(d2, axis=1, keepdims=True)
    i1 = jnp.min(jnp.where(d2 == m1, iota, _E), axis=1, keepdims=True)
    sel1 = iota == i1
    d2b = jnp.where(sel1, inf, d2)
    m2 = jnp.min(d2b, axis=1, keepdims=True)
    i2 = jnp.min(jnp.where(d2b == m2, iota, _E), axis=1, keepdims=True)
    sel2 = iota == i2
    d2c = jnp.where(sel2, inf, d2b)
    m3 = jnp.min(d2c, axis=1, keepdims=True)

    l1 = -jnp.sqrt(jnp.maximum(m1, 0.0))
    l2 = -jnp.sqrt(jnp.maximum(m2, 0.0))
    dense_ref[...] = _softmax2_dense(sel1, sel2, l1, l2)

    nr = _BTA // _TPW
    blk = pl.program_id(0)
    flag = (m3 - m2) < _TAU
    lane = lax.broadcasted_iota(jnp.int32, (nr, _TPW), 1)
    pv = jnp.where(flag.reshape(nr, _TPW), lane, _BIG)
    row_base = (blk * _BTA
                + _TPW * lax.broadcasted_iota(jnp.int32, (nr, 1), 0))
    src = _MAXU + row_base + lane
    slot_base = (row_base // _TPW) * _K
    cols = []
    for k in range(_K):
        g = jnp.min(pv, axis=1, keepdims=True)
        cols.append(row_base + jnp.where(g < _BIG, g, 0))
        src = jnp.where(lane == g, slot_base + k, src)
        pv = jnp.where(lane == g, _BIG, pv)
    idx_ref[...] = jnp.concatenate(cols, axis=1)[None]
    src_ref[...] = src[None]


def _approx_call(t1, t2, w1t, w2t):
    grid = (_TOKENS // _BTA,)
    nr = _BTA // _TPW
    return pl.pallas_call(
        _approx_block,
        grid=grid,
        in_specs=[
            pl.BlockSpec((_BTA, _D), lambda i: (i, 0)),
            pl.BlockSpec((_BTA, _D), lambda i: (i, 0)),
            pl.BlockSpec((_D, _E), lambda i: (0, 0)),
            pl.BlockSpec((_D, _E), lambda i: (0, 0)),
        ],
        out_specs=[
            pl.BlockSpec((_BTA, _E), lambda i: (i, 0)),
            pl.BlockSpec((1, nr, _K), lambda i: (i, 0, 0)),
            pl.BlockSpec((1, nr, _TPW), lambda i: (i, 0, 0)),
        ],
        out_shape=[
            jax.ShapeDtypeStruct((_TOKENS, _E), jnp.float32),
            jax.ShapeDtypeStruct((_TOKENS // _BTA, _BTA // _TPW, _K), jnp.int32),
            jax.ShapeDtypeStruct((_TOKENS // _BTA, _BTA // _TPW, _TPW), jnp.int32),
        ],
    )(t1, t2, w1t, w2t)


def _gather_kernel(idx_hbm, t1_hbm, t2_hbm, g1_hbm, g2_hbm,
                   idx_v, rows1_v, rows2_v, sem):
    wid = lax.axis_index("s") * 2 + lax.axis_index("c")
    sl = pl.ds(wid * _K, _K)
    pltpu.sync_copy(idx_hbm.at[sl], idx_v)
    pltpu.async_copy(t1_hbm.at[idx_v], rows1_v, sem).wait()
    pltpu.async_copy(t2_hbm.at[idx_v], rows2_v, sem).wait()
    pltpu.sync_copy(rows1_v, g1_hbm.at[sl])
    pltpu.sync_copy(rows2_v, g2_hbm.at[sl])


_gather = functools.partial(
    pl.kernel,
    mesh=plsc.VectorSubcoreMesh(core_axis_name="c", subcore_axis_name="s"),
    out_type=[
        jax.ShapeDtypeStruct((_MAXU, _D), jnp.float32),
        jax.ShapeDtypeStruct((_MAXU, _D), jnp.float32),
    ],
    scratch_types=[
        pltpu.VMEM((_K,), jnp.int32),
        pltpu.VMEM((_K, _D), jnp.float32),
        pltpu.VMEM((_K, _D), jnp.float32),
        pltpu.SemaphoreType.DMA,
    ],
)(_gather_kernel)


def _exact_block(g1_ref, g2_ref, w_ref, out_ref):
    rows = []
    for tg in range(_BTC // 8):
        r0 = tg * 8
        accs = [None] * _E
        for c in range((2 * _D) // _CHUNK):
            if c < _D // _CHUNK:
                xc = g1_ref[r0:r0 + 8, c * _CHUNK:(c + 1) * _CHUNK]
            else:
                cc = c - _D // _CHUNK
                xc = g2_ref[r0:r0 + 8, cc * _CHUNK:(cc + 1) * _CHUNK]
            for e in range(_E):
                w_row = w_ref[e, c * _CHUNK:(c + 1) * _CHUNK]
                diff = w_row[None, :] - xc
                sq = diff * diff
                p = jnp.sum(sq, axis=1, keepdims=True)
                accs[e] = p if c == 0 else accs[e] + p
        rows.append(jnp.concatenate(accs, axis=1))
    d2 = jnp.concatenate(rows, axis=0)
    logits = -jnp.sqrt(d2)
    sel1, sel2, m1, m2 = _top2_masks(logits)
    out_ref[...] = _softmax2_dense(sel1, sel2, m1, m2)


def _exact_call(g1, g2, W):
    grid = (_MAXU // _BTC,)
    return pl.pallas_call(
        _exact_block,
        grid=grid,
        in_specs=[
            pl.BlockSpec((_BTC, _D), lambda i: (i, 0)),
            pl.BlockSpec((_BTC, _D), lambda i: (i, 0)),
            pl.BlockSpec((_E, 2 * _D), lambda i: (0, 0)),
        ],
        out_specs=pl.BlockSpec((_BTC, _E), lambda i: (i, 0)),
        out_shape=jax.ShapeDtypeStruct((_MAXU, _E), jnp.float32),
    )(g1, g2, W)


def _merge_kernel(src_hbm, comb_hbm, out_hbm, src_v, rows_v, sem):
    wid = lax.axis_index("s") * 2 + lax.axis_index("c")
    base = wid * _TPW
    pltpu.sync_copy(src_hbm.at[pl.ds(base, _TPW)], src_v)
    pltpu.async_copy(comb_hbm.at[src_v], rows_v, sem).wait()
    pltpu.sync_copy(rows_v, out_hbm.at[pl.ds(base, _TPW)])


_merge = functools.partial(
    pl.kernel,
    mesh=plsc.VectorSubcoreMesh(core_axis_name="c", subcore_axis_name="s"),
    out_type=jax.ShapeDtypeStruct((_TOKENS, 128), jnp.float32),
    scratch_types=[
        pltpu.VMEM((_TPW,), jnp.int32),
        pltpu.VMEM((_TPW, 128), jnp.float32),
        pltpu.SemaphoreType.DMA,
    ],
)(_merge_kernel)


def kernel(tensor1, tensor2, W):
    w1t = W[:, :_D].T
    w2t = W[:, _D:].T
    dense, idx, src = _approx_call(tensor1, tensor2, w1t, w2t)
    idx_flat = idx.reshape(_MAXU)
    g1, g2 = _gather(idx_flat, tensor1, tensor2)
    rows = _exact_call(g1, g2, W)
    comb = jnp.concatenate([rows, dense], axis=0)
    comb = jnp.pad(comb, ((0, 0), (0, 128 - _E)))
    out = _merge(src.reshape(_TOKENS), comb)
    return out[:, :_E]


# speedup vs baseline: 1.3178x; 1.3178x over previous
import functools

import jax
import jax.numpy as jnp
from jax import lax
from jax.experimental import pallas as pl
from jax.experimental.pallas import tpu as pltpu
from jax.experimental.pallas import tpu_sc as plsc

_TOKENS = 4096
_D = 1024
_E = 16
_CHUNK = 128
_TAU = 0.006
_NW = 32
_TPW = _TOKENS // _NW
_K = 16
_MAXU = _NW * _K
_BTA = 512
_BTC = 64
_BIG = 1 << 20


def _top2_masks(logits):
    iota = lax.broadcasted_iota(jnp.int32, logits.shape, 1)
    m1 = jnp.max(logits, axis=1, keepdims=True)
    i1 = jnp.min(jnp.where(logits == m1, iota, _E), axis=1, keepdims=True)
    sel1 = iota == i1
    masked = jnp.where(sel1, -jnp.inf, logits)
    m2 = jnp.max(masked, axis=1, keepdims=True)
    i2 = jnp.min(jnp.where(masked == m2, iota, _E), axis=1, keepdims=True)
    sel2 = iota == i2
    return sel1, sel2, m1, m2


def _softmax2_dense(sel1, sel2, m1, m2):
    q = jnp.exp(m2 - m1)
    g1 = 1.0 / (1.0 + q)
    g2 = q / (1.0 + q)
    return jnp.where(sel1, g1, 0.0) + jnp.where(sel2, g2, 0.0)


def _dist_block(t1_ref, t2_ref, w1t_ref, w2t_ref, d2_ref):
    t1 = t1_ref[...]
    t2 = t2_ref[...]
    w1t = w1t_ref[...]
    w2t = w2t_ref[...]
    f32 = jnp.float32

    def _split(a):
        ah = a.astype(jnp.bfloat16)
        al = (a - ah.astype(f32)).astype(jnp.bfloat16)
        return ah, al

    def _dot3(a, b):
        ah, al = _split(a)
        bh, bl = _split(b)
        return (jnp.dot(ah, bh, preferred_element_type=f32)
                + (jnp.dot(ah, bl, preferred_element_type=f32)
                   + jnp.dot(al, bh, preferred_element_type=f32)))

    dot = _dot3(t1, w1t) + _dot3(t2, w2t)
    xsq = (jnp.sum(t1 * t1, axis=1, keepdims=True)
           + jnp.sum(t2 * t2, axis=1, keepdims=True))
    wsq = (jnp.sum(w1t * w1t, axis=0, keepdims=True)
           + jnp.sum(w2t * w2t, axis=0, keepdims=True))
    d2_ref[...] = xsq + wsq - 2.0 * dot


def _dist_call(t1, t2, w1t, w2t):
    grid = (_TOKENS // _BTA,)
    return pl.pallas_call(
        _dist_block,
        grid=grid,
        in_specs=[
            pl.BlockSpec((_BTA, _D), lambda i: (i, 0)),
            pl.BlockSpec((_BTA, _D), lambda i: (i, 0)),
            pl.BlockSpec((_D, _E), lambda i: (0, 0)),
            pl.BlockSpec((_D, _E), lambda i: (0, 0)),
        ],
        out_specs=pl.BlockSpec((_BTA, _E), lambda i: (i, 0)),
        out_shape=jax.ShapeDtypeStruct((_TOKENS, _E), jnp.float32),
    )(t1, t2, w1t, w2t)


def _route_block(d2_ref, dense_ref, idx_ref, src_ref):
    d2 = d2_ref[...]
    iota = lax.broadcasted_iota(jnp.int32, d2.shape, 1)
    inf = jnp.float32(jnp.inf)
    m1 = jnp.min(d2, axis=1, keepdims=True)
    i1 = jnp.min(jnp.where(d2 == m1, iota, _E), axis=1, keepdims=True)
    sel1 = iota == i1
    d2b = jnp.where(sel1, inf, d2)
    m2 = jnp.min(d2b, axis=1, keepdims=True)
    i2 = jnp.min(jnp.where(d2b == m2, iota, _E), axis=1, keepdims=True)
    sel2 = iota == i2
    d2c = jnp.where(sel2, inf, d2b)
    m3 = jnp.min(d2c, axis=1, keepdims=True)

    l1 = -jnp.sqrt(jnp.maximum(m1, 0.0))
    l2 = -jnp.sqrt(jnp.maximum(m2, 0.0))
    dense_ref[...] = _softmax2_dense(sel1, sel2, l1, l2)

    flag = (m3 - m2) < _TAU
    lane = lax.broadcasted_iota(jnp.int32, (_NW, _TPW), 1)
    pv = jnp.where(flag.reshape(_NW, _TPW), lane, _BIG)
    row_base = _TPW * lax.broadcasted_iota(jnp.int32, (_NW, 1), 0)
    src = _MAXU + row_base + lane
    slot_base = _K * lax.broadcasted_iota(jnp.int32, (_NW, 1), 0)
    cols = []
    for k in range(_K):
        g = jnp.min(pv, axis=1, keepdims=True)
        cols.append(row_base + jnp.where(g < _BIG, g, 0))
        src = jnp.where(lane == g, slot_base + k, src)
        pv = jnp.where(lane == g, _BIG, pv)
    idx_ref[...] = jnp.concatenate(cols, axis=1)[None]
    src_ref[...] = src[None]


def _route_call(d2):
    return pl.pallas_call(
        _route_block,
        out_specs=[
            pl.BlockSpec((_TOKENS, _E), lambda: (0, 0)),
            pl.BlockSpec((1, _NW, _K), lambda: (0, 0, 0)),
            pl.BlockSpec((1, _NW, _TPW), lambda: (0, 0, 0)),
        ],
        out_shape=[
            jax.ShapeDtypeStruct((_TOKENS, _E), jnp.float32),
            jax.ShapeDtypeStruct((1, _NW, _K), jnp.int32),
            jax.ShapeDtypeStruct((1, _NW, _TPW), jnp.int32),
        ],
    )(d2)


def _gather_kernel(idx_hbm, t1_hbm, t2_hbm, g1_hbm, g2_hbm,
                   idx_v, rows1_v, rows2_v, sem):
    wid = lax.axis_index("s") * 2 + lax.axis_index("c")
    sl = pl.ds(wid * _K, _K)
    pltpu.sync_copy(idx_hbm.at[sl], idx_v)
    pltpu.async_copy(t1_hbm.at[idx_v], rows1_v, sem).wait()
    pltpu.async_copy(t2_hbm.at[idx_v], rows2_v, sem).wait()
    pltpu.sync_copy(rows1_v, g1_hbm.at[sl])
    pltpu.sync_copy(rows2_v, g2_hbm.at[sl])


_gather = functools.partial(
    pl.kernel,
    mesh=plsc.VectorSubcoreMesh(core_axis_name="c", subcore_axis_name="s"),
    out_type=[
        jax.ShapeDtypeStruct((_MAXU, _D), jnp.float32),
        jax.ShapeDtypeStruct((_MAXU, _D), jnp.float32),
    ],
    scratch_types=[
        pltpu.VMEM((_K,), jnp.int32),
        pltpu.VMEM((_K, _D), jnp.float32),
        pltpu.VMEM((_K, _D), jnp.float32),
        pltpu.SemaphoreType.DMA,
    ],
)(_gather_kernel)


def _exact_block(g1_ref, g2_ref, w_ref, out_ref):
    rows = []
    for tg in range(_BTC // 8):
        r0 = tg * 8
        accs = [None] * _E
        for c in range((2 * _D) // _CHUNK):
            if c < _D // _CHUNK:
                xc = g1_ref[r0:r0 + 8, c * _CHUNK:(c + 1) * _CHUNK]
            else:
                cc = c - _D // _CHUNK
                xc = g2_ref[r0:r0 + 8, cc * _CHUNK:(cc + 1) * _CHUNK]
            for e in range(_E):
                w_row = w_ref[e, c * _CHUNK:(c + 1) * _CHUNK]
                diff = w_row[None, :] - xc
                sq = diff * diff
                p = jnp.sum(sq, axis=1, keepdims=True)
                accs[e] = p if c == 0 else accs[e] + p
        rows.append(jnp.concatenate(accs, axis=1))
    d2 = jnp.concatenate(rows, axis=0)
    logits = -jnp.sqrt(d2)
    sel1, sel2, m1, m2 = _top2_masks(logits)
    out_ref[...] = _softmax2_dense(sel1, sel2, m1, m2)


def _exact_call(g1, g2, W):
    grid = (_MAXU // _BTC,)
    return pl.pallas_call(
        _exact_block,
        grid=grid,
        in_specs=[
            pl.BlockSpec((_BTC, _D), lambda i: (i, 0)),
            pl.BlockSpec((_BTC, _D), lambda i: (i, 0)),
            pl.BlockSpec((_E, 2 * _D), lambda i: (0, 0)),
        ],
        out_specs=pl.BlockSpec((_BTC, _E), lambda i: (i, 0)),
        out_shape=jax.ShapeDtypeStruct((_MAXU, _E), jnp.float32),
    )(g1, g2, W)


def _merge_kernel(src_hbm, comb_hbm, out_hbm, src_v, rows_v, sem):
    wid = lax.axis_index("s") * 2 + lax.axis_index("c")
    base = wid * _TPW
    pltpu.sync_copy(src_hbm.at[pl.ds(base, _TPW)], src_v)
    pltpu.async_copy(comb_hbm.at[src_v], rows_v, sem).wait()
    pltpu.sync_copy(rows_v, out_hbm.at[pl.ds(base, _TPW)])


_merge = functools.partial(
    pl.kernel,
    mesh=plsc.VectorSubcoreMesh(core_axis_name="c", subcore_axis_name="s"),
    out_type=jax.ShapeDtypeStruct((_TOKENS, 128), jnp.float32),
    scratch_types=[
        pltpu.VMEM((_TPW,), jnp.int32),
        pltpu.VMEM((_TPW, 128), jnp.float32),
        pltpu.SemaphoreType.DMA,
    ],
)(_merge_kernel)


def kernel(tensor1, tensor2, W):
    w1t = W[:, :_D].T
    w2t = W[:, _D:].T
    d2 = _dist_call(tensor1, tensor2, w1t, w2t)
    dense, idx, src = _route_call(d2)
    idx_flat = idx.reshape(_MAXU)
    g1, g2 = _gather(idx_flat, tensor1, tensor2)
    rows = _exact_call(g1, g2, W)
    comb = jnp.concatenate([rows, dense], axis=0)
    comb = jnp.pad(comb, ((0, 0), (0, 128 - _E)))
    out = _merge(src.reshape(_TOKENS), comb)
    return out[:, :_E]


# speedup vs baseline: 1.6467x; 1.2495x over previous
import functools

import jax
import jax.numpy as jnp
from jax import lax
from jax.experimental import pallas as pl
from jax.experimental.pallas import tpu as pltpu
from jax.experimental.pallas import tpu_sc as plsc

_TOKENS = 4096
_D = 1024
_E = 16
_CHUNK = 128
_TAU = 0.003
_NW = 32
_TPW = _TOKENS // _NW
_K = 8
_MAXU = _NW * _K
_BTA = 512
_BTC = 64
_BIG = 1 << 20


def _top2_masks(logits):
    iota = lax.broadcasted_iota(jnp.int32, logits.shape, 1)
    m1 = jnp.max(logits, axis=1, keepdims=True)
    i1 = jnp.min(jnp.where(logits == m1, iota, _E), axis=1, keepdims=True)
    sel1 = iota == i1
    masked = jnp.where(sel1, -jnp.inf, logits)
    m2 = jnp.max(masked, axis=1, keepdims=True)
    i2 = jnp.min(jnp.where(masked == m2, iota, _E), axis=1, keepdims=True)
    sel2 = iota == i2
    return sel1, sel2, m1, m2


def _softmax2_dense(sel1, sel2, m1, m2):
    q = jnp.exp(m2 - m1)
    g1 = 1.0 / (1.0 + q)
    g2 = q / (1.0 + q)
    return jnp.where(sel1, g1, 0.0) + jnp.where(sel2, g2, 0.0)


def _dist_block(t1_ref, t2_ref, w1t_ref, w2t_ref, d2_ref):
    t1 = t1_ref[...]
    t2 = t2_ref[...]
    w1t = w1t_ref[...]
    w2t = w2t_ref[...]
    f32 = jnp.float32

    def _split(a):
        ah = a.astype(jnp.bfloat16)
        al = (a - ah.astype(f32)).astype(jnp.bfloat16)
        return ah, al

    def _dot3(a, b):
        ah, al = _split(a)
        bh, bl = _split(b)
        return (jnp.dot(ah, bh, preferred_element_type=f32)
                + (jnp.dot(ah, bl, preferred_element_type=f32)
                   + jnp.dot(al, bh, preferred_element_type=f32)))

    dot = _dot3(t1, w1t) + _dot3(t2, w2t)
    xsq = (jnp.sum(t1 * t1, axis=1, keepdims=True)
           + jnp.sum(t2 * t2, axis=1, keepdims=True))
    wsq = (jnp.sum(w1t * w1t, axis=0, keepdims=True)
           + jnp.sum(w2t * w2t, axis=0, keepdims=True))
    d2_ref[...] = xsq + wsq - 2.0 * dot


def _dist_call(t1, t2, w1t, w2t):
    grid = (_TOKENS // _BTA,)
    return pl.pallas_call(
        _dist_block,
        grid=grid,
        in_specs=[
            pl.BlockSpec((_BTA, _D), lambda i: (i, 0)),
            pl.BlockSpec((_BTA, _D), lambda i: (i, 0)),
            pl.BlockSpec((_D, _E), lambda i: (0, 0)),
            pl.BlockSpec((_D, _E), lambda i: (0, 0)),
        ],
        out_specs=pl.BlockSpec((_BTA, _E), lambda i: (i, 0)),
        out_shape=jax.ShapeDtypeStruct((_TOKENS, _E), jnp.float32),
    )(t1, t2, w1t, w2t)


def _route_block(d2_ref, dense_ref, idx_ref, src_ref):
    d2 = d2_ref[...]
    iota = lax.broadcasted_iota(jnp.int32, d2.shape, 1)
    inf = jnp.float32(jnp.inf)
    m1 = jnp.min(d2, axis=1, keepdims=True)
    i1 = jnp.min(jnp.where(d2 == m1, iota, _E), axis=1, keepdims=True)
    sel1 = iota == i1
    d2b = jnp.where(sel1, inf, d2)
    m2 = jnp.min(d2b, axis=1, keepdims=True)
    i2 = jnp.min(jnp.where(d2b == m2, iota, _E), axis=1, keepdims=True)
    sel2 = iota == i2
    d2c = jnp.where(sel2, inf, d2b)
    m3 = jnp.min(d2c, axis=1, keepdims=True)

    l1 = -jnp.sqrt(jnp.maximum(m1, 0.0))
    l2 = -jnp.sqrt(jnp.maximum(m2, 0.0))
    dense = _softmax2_dense(sel1, sel2, l1, l2)
    dense_ref[...] = jnp.pad(dense, ((0, 0), (0, 128 - _E)))

    flag = (m3 - m2) < _TAU
    lane = lax.broadcasted_iota(jnp.int32, (_NW, _TPW), 1)
    pv = jnp.where(flag.reshape(_NW, _TPW), lane, _BIG)
    row_base = _TPW * lax.broadcasted_iota(jnp.int32, (_NW, 1), 0)
    src = _MAXU + row_base + lane
    slot_base = _K * lax.broadcasted_iota(jnp.int32, (_NW, 1), 0)
    cols = []
    for k in range(_K):
        g = jnp.min(pv, axis=1, keepdims=True)
        cols.append(row_base + jnp.where(g < _BIG, g, 0))
        src = jnp.where(lane == g, slot_base + k, src)
        pv = jnp.where(lane == g, _BIG, pv)
    idx_ref[...] = jnp.concatenate(cols, axis=1)[None]
    src_ref[...] = src[None]


def _route_call(d2):
    return pl.pallas_call(
        _route_block,
        grid=(1,),
        in_specs=[pl.BlockSpec((_TOKENS, _E), lambda i: (0, 0))],
        out_specs=[
            pl.BlockSpec((_TOKENS, 128), lambda i: (_MAXU // _TOKENS, 0)),
            pl.BlockSpec((1, _NW, _K), lambda i: (0, 0, 0)),
            pl.BlockSpec((1, _NW, _TPW), lambda i: (0, 0, 0)),
        ],
        out_shape=[
            jax.ShapeDtypeStruct((_MAXU + _TOKENS, 128), jnp.float32),
            jax.ShapeDtypeStruct((1, _NW, _K), jnp.int32),
            jax.ShapeDtypeStruct((1, _NW, _TPW), jnp.int32),
        ],
    )(d2)


def _gather_kernel(idx_hbm, t1_hbm, t2_hbm, g1_hbm, g2_hbm,
                   idx_v, rows1_v, rows2_v, sem):
    wid = lax.axis_index("s") * 2 + lax.axis_index("c")
    sl = pl.ds(wid * _K, _K)
    pltpu.sync_copy(idx_hbm.at[sl], idx_v)
    pltpu.async_copy(t1_hbm.at[idx_v], rows1_v, sem).wait()
    pltpu.async_copy(t2_hbm.at[idx_v], rows2_v, sem).wait()
    pltpu.sync_copy(rows1_v, g1_hbm.at[sl])
    pltpu.sync_copy(rows2_v, g2_hbm.at[sl])


_gather = functools.partial(
    pl.kernel,
    mesh=plsc.VectorSubcoreMesh(core_axis_name="c", subcore_axis_name="s"),
    out_type=[
        jax.ShapeDtypeStruct((_MAXU, _D), jnp.float32),
        jax.ShapeDtypeStruct((_MAXU, _D), jnp.float32),
    ],
    scratch_types=[
        pltpu.VMEM((_K,), jnp.int32),
        pltpu.VMEM((_K, _D), jnp.float32),
        pltpu.VMEM((_K, _D), jnp.float32),
        pltpu.SemaphoreType.DMA,
    ],
)(_gather_kernel)


def _exact_block(g1_ref, g2_ref, w_ref, comb_ref, out_ref):
    rows = []
    for tg in range(_BTC // 8):
        r0 = tg * 8
        accs = [None] * _E
        for c in range((2 * _D) // _CHUNK):
            if c < _D // _CHUNK:
                xc = g1_ref[r0:r0 + 8, c * _CHUNK:(c + 1) * _CHUNK]
            else:
                cc = c - _D // _CHUNK
                xc = g2_ref[r0:r0 + 8, cc * _CHUNK:(cc + 1) * _CHUNK]
            for e in range(_E):
                w_row = w_ref[e, c * _CHUNK:(c + 1) * _CHUNK]
                diff = w_row[None, :] - xc
                sq = diff * diff
                p = jnp.sum(sq, axis=1, keepdims=True)
                accs[e] = p if c == 0 else accs[e] + p
        rows.append(jnp.concatenate(accs, axis=1))
    del comb_ref
    d2 = jnp.concatenate(rows, axis=0)
    logits = -jnp.sqrt(d2)
    sel1, sel2, m1, m2 = _top2_masks(logits)
    out_ref[...] = jnp.pad(_softmax2_dense(sel1, sel2, m1, m2),
                           ((0, 0), (0, 128 - _E)))


def _exact_call(g1, g2, W, comb):
    grid = (_MAXU // _BTC,)
    return pl.pallas_call(
        _exact_block,
        grid=grid,
        in_specs=[
            pl.BlockSpec((_BTC, _D), lambda i: (i, 0)),
            pl.BlockSpec((_BTC, _D), lambda i: (i, 0)),
            pl.BlockSpec((_E, 2 * _D), lambda i: (0, 0)),
            pl.BlockSpec(memory_space=pl.ANY),
        ],
        out_specs=pl.BlockSpec((_BTC, 128), lambda i: (i, 0)),
        out_shape=jax.ShapeDtypeStruct((_MAXU + _TOKENS, 128), jnp.float32),
        input_output_aliases={3: 0},
    )(g1, g2, W, comb)


def _merge_kernel(src_hbm, comb_hbm, out_hbm, src_v, rows_v, sem):
    wid = lax.axis_index("s") * 2 + lax.axis_index("c")
    base = wid * _TPW
    pltpu.sync_copy(src_hbm.at[pl.ds(base, _TPW)], src_v)
    pltpu.async_copy(comb_hbm.at[src_v], rows_v, sem).wait()
    pltpu.sync_copy(rows_v, out_hbm.at[pl.ds(base, _TPW)])


_merge = functools.partial(
    pl.kernel,
    mesh=plsc.VectorSubcoreMesh(core_axis_name="c", subcore_axis_name="s"),
    out_type=jax.ShapeDtypeStruct((_TOKENS, 128), jnp.float32),
    scratch_types=[
        pltpu.VMEM((_TPW,), jnp.int32),
        pltpu.VMEM((_TPW, 128), jnp.float32),
        pltpu.SemaphoreType.DMA,
    ],
)(_merge_kernel)


def kernel(tensor1, tensor2, W):
    w1t = W[:, :_D].T
    w2t = W[:, _D:].T
    d2 = _dist_call(tensor1, tensor2, w1t, w2t)
    comb0, idx, src = _route_call(d2)
    idx_flat = idx.reshape(_MAXU)
    g1, g2 = _gather(idx_flat, tensor1, tensor2)
    comb = _exact_call(g1, g2, W, comb0)
    out = _merge(src.reshape(_TOKENS), comb)
    return out[:, :_E]
